# baseline (speedup 1.0000x reference)
import functools
import math

import jax
import jax.numpy as jnp
from jax.experimental import pallas as pl
from jax.experimental.pallas import tpu as pltpu

_BE = 2000
_BN = 2000


def _l2n(x, axis=-1, eps=1e-12):
    n = jnp.linalg.norm(x, axis=axis, keepdims=True)
    return x / jnp.maximum(n, eps)


def _orientations(coords):
    N = coords.shape[0]
    X = coords.reshape(1, 3 * N, 3)
    dX = X[:, 1:, :] - X[:, :-1, :]
    U = _l2n(dX)
    u_2 = U[:, :-2, :]
    u_1 = U[:, 1:-1, :]
    n_2 = _l2n(jnp.cross(u_2, u_1))
    o_1 = _l2n(u_2 - u_1)
    O = jnp.stack((o_1, n_2, jnp.cross(o_1, n_2)), axis=2)
    O = O.reshape(1, 3 * N - 3, 9)
    O = jnp.pad(O, ((0, 0), (1, 2), (0, 0)))
    O = O.reshape(N, 3, 9)
    return O[:, 1, :]


_BQ = _BE // 8


def _gather_body(idx_ref, table_ref, out_ref):
    def body(i, _):
        j = i // _BQ
        k = i - j * _BQ
        r = idx_ref[0, j, k]
        out_ref[pl.ds(i, 1), :] = table_ref[pl.ds(r, 1), :]
        return 0
    jax.lax.fori_loop(0, out_ref.shape[0], body, 0)


def _gather(table, idx):
    E = idx.shape[0]
    N, C = table.shape
    return pl.pallas_call(
        _gather_body,
        grid=(E // _BE,),
        in_specs=[
            pl.BlockSpec((1, 8, _BQ), lambda i: (i, 0, 0),
                         memory_space=pltpu.SMEM),
            pl.BlockSpec((N, C), lambda i: (0, 0)),
        ],
        out_specs=pl.BlockSpec((_BE, C), lambda i: (i, 0)),
        out_shape=jax.ShapeDtypeStruct((E, C), table.dtype),
    )(idx.reshape(E // _BE, 8, _BQ), table)


def _scatter_body(idx_ref, vals_ref, out_ref):
    @pl.when(pl.program_id(0) == 0)
    def _():
        out_ref[...] = jnp.zeros_like(out_ref)

    def body(i, _):
        j = i // _BQ
        k = i - j * _BQ
        r = idx_ref[0, j, k]
        cur = out_ref[pl.ds(r, 1), :]
        out_ref[pl.ds(r, 1), :] = cur + vals_ref[pl.ds(i, 1), :]
        return 0
    jax.lax.fori_loop(0, vals_ref.shape[0], body, 0)


def _scatter_add(vals, idx, n):
    E, C = vals.shape
    return pl.pallas_call(
        _scatter_body,
        grid=(E // _BE,),
        in_specs=[
            pl.BlockSpec((1, 8, _BQ), lambda i: (i, 0, 0),
                         memory_space=pltpu.SMEM),
            pl.BlockSpec((_BE, C), lambda i: (i, 0)),
        ],
        out_specs=pl.BlockSpec((n, C), lambda i: (0, 0)),
        out_shape=jax.ShapeDtypeStruct((n, C), vals.dtype),
    )(idx.reshape(E // _BE, 8, _BQ), vals)


def _scatter_max_body(idx_ref, vals_ref, out_ref):
    @pl.when(pl.program_id(0) == 0)
    def _():
        out_ref[...] = jnp.full_like(out_ref, -jnp.inf)

    def body(i, _):
        j = i // _BQ
        k = i - j * _BQ
        r = idx_ref[0, j, k]
        cur = out_ref[pl.ds(r, 1), :]
        out_ref[pl.ds(r, 1), :] = jnp.maximum(cur, vals_ref[pl.ds(i, 1), :])
        return 0
    jax.lax.fori_loop(0, vals_ref.shape[0], body, 0)

    @pl.when(pl.program_id(0) == pl.num_programs(0) - 1)
    def _():
        m = out_ref[...]
        out_ref[...] = jnp.where(jnp.isfinite(m), m, 0.0)


def _scatter_max(vals, idx, n):
    E, C = vals.shape
    return pl.pallas_call(
        _scatter_max_body,
        grid=(E // _BE,),
        in_specs=[
            pl.BlockSpec((1, 8, _BQ), lambda i: (i, 0, 0),
                         memory_space=pltpu.SMEM),
            pl.BlockSpec((_BE, C), lambda i: (i, 0)),
        ],
        out_specs=pl.BlockSpec((n, C), lambda i: (0, 0)),
        out_shape=jax.ShapeDtypeStruct((n, C), vals.dtype),
    )(idx.reshape(E // _BE, 8, _BQ), vals)


def _mm_body(x_ref, w_ref, b_ref, o_ref):
    o_ref[...] = (
        jnp.dot(x_ref[...], w_ref[...], preferred_element_type=jnp.float32)
        + b_ref[...]
    )


def _matmul(x, w, b, blk):
    M, K = x.shape
    C = w.shape[1]
    return pl.pallas_call(
        _mm_body,
        grid=(M // blk,),
        in_specs=[
            pl.BlockSpec((blk, K), lambda i: (i, 0)),
            pl.BlockSpec((K, C), lambda i: (0, 0)),
            pl.BlockSpec((1, C), lambda i: (0, 0)),
        ],
        out_specs=pl.BlockSpec((blk, C), lambda i: (i, 0)),
        out_shape=jax.ShapeDtypeStruct((M, C), jnp.float32),
    )(x, w, b)


def _feat_body(gs_ref, gd_ref, sf_ref, df_ref, o_ref):
    gs = gs_ref[...]
    gd = gd_ref[...]
    nl = gs[:, 0:20] - gd[:, 0:20]
    diff = gs[:, 20:29] - gd[:, 20:29]
    rbfs = []
    rvs = []
    for g in range(3):
        d = diff[:, 3 * g:3 * g + 3]
        n = jnp.sqrt(jnp.sum(d * d, axis=1, keepdims=True))
        rbfs.append(jnp.exp(-n))
        rvs.append(d / jnp.maximum(n, 1e-12))
    rbf = jnp.concatenate(rbfs, axis=1)
    rvec = jnp.concatenate(rvs, axis=1)
    os_ = gs[:, 29:38]
    od_ = gd[:, 29:38]
    ocols = []
    for i in range(3):
        for j in range(3):
            ocols.append(
                os_[:, i:i + 1] * od_[:, j:j + 1]
                + os_[:, 3 + i:4 + i] * od_[:, 3 + j:4 + j]
                + os_[:, 6 + i:7 + i] * od_[:, 6 + j:7 + j]
            )
    orient = jnp.concatenate(ocols, axis=1)
    v36 = rvec[:, 3:6]
    vec = v36 / jnp.maximum(jnp.abs(v36), 1e-12)
    ovs = []
    for i in range(3):
        ovs.append(
            os_[:, i:i + 1] * vec[:, 0:1]
            + os_[:, 3 + i:4 + i] * vec[:, 1:2]
            + os_[:, 6 + i:7 + i] * vec[:, 2:3]
        )
    ov = jnp.concatenate(ovs, axis=1)
    o_ref[...] = jnp.concatenate(
        [sf_ref[...] - df_ref[...], nl, rbf, rvec, orient, ov], axis=1)


def _edge_features(gsrc, gdst, srcf, dstf):
    E = gsrc.shape[0]
    return pl.pallas_call(
        _feat_body,
        grid=(E // _BE,),
        in_specs=[
            pl.BlockSpec((_BE, 38), lambda i: (i, 0)),
            pl.BlockSpec((_BE, 38), lambda i: (i, 0)),
            pl.BlockSpec((_BE, 1), lambda i: (i, 0)),
            pl.BlockSpec((_BE, 1), lambda i: (i, 0)),
        ],
        out_specs=pl.BlockSpec((_BE, 45), lambda i: (i, 0)),
        out_shape=jax.ShapeDtypeStruct((E, 45), jnp.float32),
    )(gsrc, gdst, srcf, dstf)


def _alpha_body(kv_ref, q_ref, ep_ref, a_ref, *, C):
    k_e = kv_ref[:, 0:C] + ep_ref[...]
    al = jnp.sum(q_ref[...] * k_e, axis=1, keepdims=True) / math.sqrt(C)
    a_ref[...] = al


def _alpha(kv, q, ep):
    E, C = q.shape
    return pl.pallas_call(
        functools.partial(_alpha_body, C=C),
        grid=(E // _BE,),
        in_specs=[
            pl.BlockSpec((_BE, 2 * C), lambda i: (i, 0)),
            pl.BlockSpec((_BE, C), lambda i: (i, 0)),
            pl.BlockSpec((_BE, C), lambda i: (i, 0)),
        ],
        out_specs=pl.BlockSpec((_BE, 1), lambda i: (i, 0)),
        out_shape=jax.ShapeDtypeStruct((E, 1), jnp.float32),
    )(kv, q, ep)


def _exv_body(al_ref, m_ref, kv_ref, ep_ref, o_ref, *, C):
    v_e = kv_ref[:, C:2 * C] + ep_ref[...]
    ex = jnp.exp(al_ref[...] - m_ref[...])
    o_ref[...] = jnp.concatenate([ex * v_e, ex], axis=1)


def _exv(al, m, kv, ep):
    E, C = ep.shape
    return pl.pallas_call(
        functools.partial(_exv_body, C=C),
        grid=(E // _BE,),
        in_specs=[
            pl.BlockSpec((_BE, 1), lambda i: (i, 0)),
            pl.BlockSpec((_BE, 1), lambda i: (i, 0)),
            pl.BlockSpec((_BE, 2 * C), lambda i: (i, 0)),
            pl.BlockSpec((_BE, C), lambda i: (i, 0)),
        ],
        out_specs=pl.BlockSpec((_BE, C + 1), lambda i: (i, 0)),
        out_shape=jax.ShapeDtypeStruct((E, C + 1), jnp.float32),
    )(al, m, kv, ep)


def _final_body(s_ref, x_ref, w_ref, b_ref, o_ref, *, C, act):
    agg = s_ref[:, 0:C] / (s_ref[:, C:C + 1] + 1e-16)
    h = agg + jnp.dot(
        x_ref[...], w_ref[...], preferred_element_type=jnp.float32
    ) + b_ref[...]
    if act == 0:
        h = jnp.maximum(h, 0.0)
    elif act == 1:
        h = jax.nn.sigmoid(h)
    o_ref[...] = h


def _final(s, x, w, b, act):
    N, K = x.shape
    C = w.shape[1]
    return pl.pallas_call(
        functools.partial(_final_body, C=C, act=act),
        grid=(N // _BN,),
        in_specs=[
            pl.BlockSpec((_BN, C + 1), lambda i: (i, 0)),
            pl.BlockSpec((_BN, K), lambda i: (i, 0)),
            pl.BlockSpec((K, C), lambda i: (0, 0)),
            pl.BlockSpec((1, C), lambda i: (0, 0)),
        ],
        out_specs=pl.BlockSpec((_BN, C), lambda i: (i, 0)),
        out_shape=jax.ShapeDtypeStruct((N, C), jnp.float32),
    )(s, x, w, b)


def _layer(x, src, dst, edge_feat, Wq, bq, Wk, bk, Wv, bv, We, Ws, bs, act):
    N = x.shape[0]
    C = Wq.shape[1]
    Wqkv = jnp.concatenate([Wq, Wk, Wv], axis=1)
    bqkv = jnp.concatenate([bq, bk, bv]).reshape(1, 3 * C)
    qkv = _matmul(x, Wqkv, bqkv, _BN)
    ep = _matmul(edge_feat, We, jnp.zeros((1, C), jnp.float32), _BE)
    kv_src = _gather(qkv[:, C:], src)
    q_dst = _gather(qkv[:, :C], dst)
    al = _alpha(kv_src, q_dst, ep)
    m = _scatter_max(al, dst, N)
    mg = _gather(m, dst)
    p = _exv(al, mg, kv_src, ep)
    s = _scatter_add(p, dst, N)
    return _final(s, x, Ws, bs.reshape(1, C), act)


def kernel(t, data, edge_index, amino_index, L0_Wq, L0_Wk, L0_Wv, L0_We, L0_Ws, L0_bq, L0_bk, L0_bv, L0_bs, L1_Wq, L1_Wk, L1_Wv, L1_We, L1_Ws, L1_bq, L1_bk, L1_bv, L1_bs, L2_Wq, L2_Wk, L2_Wv, L2_We, L2_Ws, L2_bq, L2_bk, L2_bv, L2_bs, L3_Wq, L3_Wk, L3_Wv, L3_We, L3_Ws, L3_bq, L3_bk, L3_bv, L3_bs):
    N = data.shape[0]
    src = edge_index[0]
    dst = edge_index[1]
    coords = data[:, 20:29].reshape(N, 3, 3)
    O = _orientations(coords)
    t38 = jnp.concatenate([data[:, :29], O], axis=1)
    gsrc = _gather(t38, src)
    gdst = _gather(t38, dst)
    amino = amino_index.astype(jnp.float32).reshape(-1, 1)
    srcf = _gather(amino, src)
    dstf = _gather(amino, dst)
    edge_feat = _edge_features(gsrc, gdst, srcf, dstf)
    h = jnp.concatenate([jnp.ones((N, 1), jnp.float32) * t, data], axis=1)
    layers = [
        (L0_Wq, L0_bq, L0_Wk, L0_bk, L0_Wv, L0_bv, L0_We, L0_Ws, L0_bs, 0),
        (L1_Wq, L1_bq, L1_Wk, L1_bk, L1_Wv, L1_bv, L1_We, L1_Ws, L1_bs, 1),
        (L2_Wq, L2_bq, L2_Wk, L2_bk, L2_Wv, L2_bv, L2_We, L2_Ws, L2_bs, 1),
        (L3_Wq, L3_bq, L3_Wk, L3_bk, L3_Wv, L3_bv, L3_We, L3_Ws, L3_bs, 2),
    ]
    for (Wq, bq, Wk, bk, Wv, bv, We, Ws, bs, act) in layers:
        h = _layer(h, src, dst, edge_feat, Wq, bq, Wk, bk, Wv, bv, We, Ws,
                   bs, act)
    return h
